# baseline (speedup 1.0000x reference)
import functools

import jax
import jax.numpy as jnp
from jax import lax
from jax.experimental import pallas as pl
from jax.experimental.pallas import tpu as pltpu
from jax.experimental.pallas import tpu_sc as plsc

_NC = 2
_NS = 16
_NW = _NC * _NS
_L = 16


@functools.lru_cache(maxsize=None)
def _sc_lookup(n_rows: int, n_cols: int, table_n: int, cblk: int):
    assert n_cols % (_NW * cblk) == 0 and cblk % _L == 0
    steps = n_cols // (_NW * cblk)
    cols_per_w = steps * cblk
    vecs_per_row = cblk // _L
    mesh = plsc.VectorSubcoreMesh(core_axis_name="c", subcore_axis_name="s")

    @functools.partial(
        pl.kernel,
        mesh=mesh,
        out_type=jax.ShapeDtypeStruct((n_rows, n_cols), jnp.int32),
        scratch_types=[
            pltpu.VMEM((table_n,), jnp.int32),
            pltpu.VMEM((n_rows, cblk), jnp.int32),
            pltpu.VMEM((n_rows, cblk), jnp.int32),
            pltpu.VMEM((n_rows, cblk), jnp.int32),
            pltpu.VMEM((n_rows, cblk), jnp.int32),
            pltpu.SemaphoreType.DMA,
            pltpu.SemaphoreType.DMA,
            pltpu.SemaphoreType.DMA,
            pltpu.SemaphoreType.DMA,
        ],
        compiler_params=pltpu.CompilerParams(needs_layout_passes=False),
    )
    def k(idx_hbm, table_hbm, out_hbm, table_v,
          bin0, bin1, bout0, bout1, sin0, sin1, sout0, sout1):
        wid = lax.axis_index("s") * _NC + lax.axis_index("c")
        pltpu.sync_copy(table_hbm, table_v)
        col0 = pl.multiple_of(wid * cols_per_w, 8)
        bins, bouts = (bin0, bin1), (bout0, bout1)
        sins, souts = (sin0, sin1), (sout0, sout1)

        def start_in(s):
            base = pl.multiple_of(col0 + s * cblk, 8)
            return pltpu.async_copy(
                idx_hbm.at[:, pl.ds(base, cblk)], bins[s % 2], sins[s % 2]
            )

        in_h, out_h = {}, {}
        in_h[0] = start_in(0)
        for s in range(steps):
            if s + 1 < steps:
                in_h[s + 1] = start_in(s + 1)
            in_h[s].wait()
            if s >= 2:
                out_h[s - 2].wait()
            b_i, b_o = bins[s % 2], bouts[s % 2]

            def body(r, _, b_i=b_i, b_o=b_o):
                for v in range(vecs_per_row):
                    idx = b_i[r, pl.ds(v * _L, _L)]
                    b_o[r, pl.ds(v * _L, _L)] = plsc.load_gather(
                        table_v, [idx]
                    )
                return 0

            lax.fori_loop(0, n_rows, body, 0, unroll=2)
            base = pl.multiple_of(col0 + s * cblk, 8)
            out_h[s] = pltpu.async_copy(
                b_o, out_hbm.at[:, pl.ds(base, cblk)], souts[s % 2]
            )
        for s in range(max(0, steps - 2), steps):
            out_h[s].wait()

    return k


def kernel(inputs, table):
    idx_t = inputs.astype(jnp.int32).T
    out_t = _sc_lookup(idx_t.shape[0], idx_t.shape[1], table.shape[0], 128)(
        idx_t, table
    )
    return out_t.T


# speedup vs baseline: 1.9578x; 1.9578x over previous
import functools

import jax
import jax.numpy as jnp
from jax import lax
from jax.experimental import pallas as pl
from jax.experimental.pallas import tpu as pltpu
from jax.experimental.pallas import tpu_sc as plsc

_NC = 2
_NS = 16
_NW = _NC * _NS
_L = 16


@functools.lru_cache(maxsize=None)
def _sc_lookup(n_rows: int, n_cols: int, table_n: int, cblk: int):
    assert n_cols % (_NW * cblk) == 0 and cblk % _L == 0
    steps = n_cols // (_NW * cblk)
    cols_per_w = steps * cblk
    vecs_per_row = cblk // _L
    mesh = plsc.VectorSubcoreMesh(core_axis_name="c", subcore_axis_name="s")

    @functools.partial(
        pl.kernel,
        mesh=mesh,
        out_type=jax.ShapeDtypeStruct((n_rows, n_cols), jnp.int32),
        scratch_types=[
            pltpu.VMEM((table_n,), jnp.int32),
            pltpu.VMEM((n_rows, cblk), jnp.int32),
            pltpu.VMEM((n_rows, cblk), jnp.int32),
        ],
        compiler_params=pltpu.CompilerParams(needs_layout_passes=False),
    )
    def k(idx_hbm, table_hbm, out_hbm, table_v, bin_v, bout_v):
        wid = lax.axis_index("s") * _NC + lax.axis_index("c")
        pltpu.sync_copy(table_hbm, table_v)
        col0 = pl.multiple_of(wid * cols_per_w, 8)

        def step(s, _):
            base = pl.multiple_of(col0 + s * cblk, 8)
            pltpu.sync_copy(idx_hbm.at[:, pl.ds(base, cblk)], bin_v)

            @plsc.parallel_loop(0, n_rows, unroll=4)
            def body(r):
                for v in range(vecs_per_row):
                    idx = bin_v[r, pl.ds(v * _L, _L)]
                    bout_v[r, pl.ds(v * _L, _L)] = plsc.load_gather(
                        table_v, [idx]
                    )

            pltpu.sync_copy(bout_v, out_hbm.at[:, pl.ds(base, cblk)])
            return 0

        lax.fori_loop(0, steps, step, 0)

    return k


def kernel(inputs, table):
    idx_t = inputs.astype(jnp.int32).T
    out_t = _sc_lookup(idx_t.shape[0], idx_t.shape[1], table.shape[0], 128)(
        idx_t, table
    )
    return out_t.T


# speedup vs baseline: 2.2826x; 1.1659x over previous
import functools

import jax
import jax.numpy as jnp
from jax import lax
from jax.experimental import pallas as pl
from jax.experimental.pallas import tpu as pltpu
from jax.experimental.pallas import tpu_sc as plsc

_NC = 2
_NS = 16
_NW = _NC * _NS
_L = 16


@functools.lru_cache(maxsize=None)
def _sc_lookup(n_rows: int, n_cols: int, table_n: int, cblk: int):
    assert n_cols % (_NW * cblk) == 0 and cblk % _L == 0
    steps = n_cols // (_NW * cblk)
    cols_per_w = steps * cblk
    vecs_per_row = cblk // _L
    mesh = plsc.VectorSubcoreMesh(core_axis_name="c", subcore_axis_name="s")

    @functools.partial(
        pl.kernel,
        mesh=mesh,
        out_type=jax.ShapeDtypeStruct((n_rows, n_cols), jnp.int32),
        scratch_types=[
            pltpu.VMEM((table_n,), jnp.int32),
            pltpu.VMEM((n_rows, cblk), jnp.int32),
            pltpu.VMEM((n_rows, cblk), jnp.int32),
            pltpu.VMEM((n_rows, cblk), jnp.int32),
            pltpu.VMEM((n_rows, cblk), jnp.int32),
            pltpu.SemaphoreType.DMA,
            pltpu.SemaphoreType.DMA,
            pltpu.SemaphoreType.DMA,
            pltpu.SemaphoreType.DMA,
        ],
        compiler_params=pltpu.CompilerParams(needs_layout_passes=False),
    )
    def k(idx_hbm, table_hbm, out_hbm, table_v,
          bin0, bin1, bout0, bout1, sin0, sin1, sout0, sout1):
        wid = lax.axis_index("s") * _NC + lax.axis_index("c")
        pltpu.sync_copy(table_hbm, table_v)
        col0 = pl.multiple_of(wid * cols_per_w, 8)
        bins, bouts = (bin0, bin1), (bout0, bout1)
        sins, souts = (sin0, sin1), (sout0, sout1)

        def start_in(s):
            base = pl.multiple_of(col0 + s * cblk, 8)
            return pltpu.async_copy(
                idx_hbm.at[:, pl.ds(base, cblk)], bins[s % 2], sins[s % 2]
            )

        in_h, out_h = {}, {}
        in_h[0] = start_in(0)
        for s in range(steps):
            if s + 1 < steps:
                in_h[s + 1] = start_in(s + 1)
            in_h[s].wait()
            if s >= 2:
                out_h[s - 2].wait()
            b_i, b_o = bins[s % 2], bouts[s % 2]

            @plsc.parallel_loop(0, n_rows, unroll=4)
            def body(r, b_i=b_i, b_o=b_o):
                for v in range(vecs_per_row):
                    idx = b_i[r, pl.ds(v * _L, _L)]
                    b_o[r, pl.ds(v * _L, _L)] = plsc.load_gather(
                        table_v, [idx]
                    )

            base = pl.multiple_of(col0 + s * cblk, 8)
            out_h[s] = pltpu.async_copy(
                b_o, out_hbm.at[:, pl.ds(base, cblk)], souts[s % 2]
            )
        for s in range(max(0, steps - 2), steps):
            out_h[s].wait()

    return k


def kernel(inputs, table):
    idx_t = inputs.astype(jnp.int32).T
    out_t = _sc_lookup(idx_t.shape[0], idx_t.shape[1], table.shape[0], 128)(
        idx_t, table
    )
    return out_t.T
